# baseline (speedup 1.0000x reference)
import jax
import jax.numpy as jnp
from jax.experimental import pallas as pl
from jax.experimental.pallas import tpu as pltpu

CODE_NUM = 2048
CODE_SIZE = 48
GRAPH_SIZE = 32
HIDDEN = 64
ATT = 32
CATE_NUM = 128
OUT_SIZE = 2048
TEXT = 300
B = 8
T = 4
BT = B * T
NBLK = 4
BLK = CODE_NUM // NBLK
ZROWS = T * CODE_SIZE


def _fused_kernel(cxa_ref, nba_ref, dv_ref, cate_ref, adj_ref, Ct_ref,
                  Nt_ref, Ut_ref, CEt_ref, cadjT_ref, WgT_ref, bg_ref,
                  WcT_ref, bc_ref, xt_ref, zt_ref, st_ref, no_ref):
    s = pl.program_id(0)
    j = pl.program_id(1)

    @pl.when((s == 0) & (j == 0))
    def _build_first():
        Ctv = Ct_ref[...]
        Ntv = Nt_ref[...]
        for t in range(T):
            c = cxa_ref[t:t + 1, :]
            n = nba_ref[t:t + 1, :]
            zt_ref[t * CODE_SIZE:(t + 1) * CODE_SIZE, :] = Ctv * c + Ntv * n

    @pl.when(s < B)
    def _matmul():
        par = jax.lax.rem(s, 2)
        ztv = zt_ref[pl.ds(par * ZROWS, ZROWS), :]
        adj_blk = adj_ref[pl.ds(j * BLK, BLK), :]
        st_ref[pl.ds(par * ZROWS, ZROWS), pl.ds(j * BLK, BLK)] = (
            jax.lax.dot_general(ztv, adj_blk, (((1,), (1,)), ((), ())),
                                preferred_element_type=jnp.float32))

    @pl.when((j == NBLK - 1) & (s < B - 1))
    def _build_next():
        par = jax.lax.rem(s + 1, 2)
        Ctv = Ct_ref[...]
        Ntv = Nt_ref[...]
        for t in range(T):
            c = cxa_ref[pl.ds((s + 1) * T + t, 1), :]
            n = nba_ref[pl.ds((s + 1) * T + t, 1), :]
            zt_ref[pl.ds(par * ZROWS + t * CODE_SIZE, CODE_SIZE), :] = (
                Ctv * c + Ntv * n)

    @pl.when(s >= 1)
    def _graph():
        par = jax.lax.rem(s - 1, 2)
        c = cxa_ref[pl.ds((s - 1) * T + j, 1), :]
        n = nba_ref[pl.ds((s - 1) * T + j, 1), :]
        Sbt = st_ref[pl.ds(par * ZROWS + j * CODE_SIZE, CODE_SIZE), :]
        coT = jnp.tanh(jnp.dot(WgT_ref[...], c * (Ct_ref[...] + Sbt),
                               preferred_element_type=jnp.float32)
                       + bg_ref[...])
        noT = jnp.tanh(jnp.dot(WgT_ref[...], n * (Nt_ref[...] + Sbt),
                               preferred_element_type=jnp.float32)
                       + bg_ref[...])
        m1 = dv_ref[0, 0, 0:1, :]
        m23 = dv_ref[0, 0, 1:2, :] + dv_ref[0, 0, 2:3, :]
        pers = jnp.sum(coT * m1, axis=1, keepdims=True)
        candT = m23 * no_ref[...]
        sc = jnp.sum(candT * Ut_ref[...], axis=0, keepdims=True)
        mx = jnp.max(sc)
        ex = jnp.exp(sc - mx)
        emer0 = jnp.sum(candT * ex, axis=1, keepdims=True) / jnp.sum(ex)
        emer = jnp.where(j == 0, 0.0, emer0)
        cate_row = cate_ref[0, 0:1, :]
        ccenT = CEt_ref[...] * cate_row
        cc2T = jnp.dot(ccenT, cadjT_ref[...],
                       preferred_element_type=jnp.float32)
        caoT = jnp.tanh(jnp.dot(WcT_ref[...], ccenT + cate_row * cc2T,
                                preferred_element_type=jnp.float32)
                        + bc_ref[...])
        pool = (jnp.sum(caoT * cate_row, axis=1, keepdims=True)
                / (jnp.sum(cate_row) + 1e-6))
        xt_ref[0, 0:GRAPH_SIZE, 0:1] = pers
        xt_ref[0, GRAPH_SIZE:2 * GRAPH_SIZE, 0:1] = emer
        xt_ref[0, 2 * GRAPH_SIZE:3 * GRAPH_SIZE, 0:1] = pool
        no_ref[...] = noT


def _head_kernel(x_ref, lens_ref, user_ref, tf_ref, eg_ref, ea_ref, ec_ref,
                 Wz_ref, Uz_ref, bz_ref, Wr_ref, Ur_ref, br_ref,
                 Wh_ref, Uh_ref, bh_ref, Wout_ref, bout_ref,
                 Wa_ref, ba_ref, va_ref, gam_ref, bet_ref,
                 Wcls_ref, bcls_ref, out_ref):
    f32 = jnp.float32
    x = x_ref[...]
    h = jnp.zeros((B, HIDDEN), f32)
    Vs = []
    for t in range(T):
        xt = x[:, t, :]
        z = jax.nn.sigmoid(jnp.dot(xt, Wz_ref[...], preferred_element_type=f32)
                           + jnp.dot(h, Uz_ref[...], preferred_element_type=f32)
                           + bz_ref[...])
        r = jax.nn.sigmoid(jnp.dot(xt, Wr_ref[...], preferred_element_type=f32)
                           + jnp.dot(h, Ur_ref[...], preferred_element_type=f32)
                           + br_ref[...])
        hh = jnp.tanh(jnp.dot(xt, Wh_ref[...], preferred_element_type=f32)
                      + jnp.dot(r * h, Uh_ref[...], preferred_element_type=f32)
                      + bh_ref[...])
        h = (1.0 - z) * h + z * hh
        Vs.append(jnp.tanh(jnp.dot(h, Wout_ref[...], preferred_element_type=f32)
                           + bout_ref[...]))
    scs = []
    for t in range(T):
        u = jnp.tanh(jnp.dot(Vs[t], Wa_ref[...], preferred_element_type=f32)
                     + ba_ref[...])
        scs.append(jnp.dot(u, va_ref[...], preferred_element_type=f32))
    sc = jnp.concatenate(scs, axis=1)
    len_c = jnp.maximum(lens_ref[...], 1)
    tio = jax.lax.broadcasted_iota(jnp.int32, (B, T), 1)
    sc = jnp.where(tio < len_c, sc, -jnp.inf)
    mx = jnp.max(sc, axis=1, keepdims=True)
    ex = jnp.exp(sc - mx)
    al = ex / jnp.sum(ex, axis=1, keepdims=True)
    pooled = jnp.zeros((B, HIDDEN), f32)
    for t in range(T):
        pooled = pooled + al[:, t:t + 1] * Vs[t]
    g = user_ref[...]
    oh1 = (jax.lax.broadcasted_iota(jnp.int32, (B, 2), 1)
           == g[:, 0:1]).astype(f32)
    oh2 = (jax.lax.broadcasted_iota(jnp.int32, (B, 9), 1)
           == g[:, 1:2]).astype(f32)
    oh3 = (jax.lax.broadcasted_iota(jnp.int32, (B, 20), 1)
           == g[:, 2:3]).astype(f32)
    u1 = jnp.dot(oh1, eg_ref[...], preferred_element_type=f32)
    u2 = jnp.dot(oh2, ea_ref[...], preferred_element_type=f32)
    u3 = jnp.dot(oh3, ec_ref[...], preferred_element_type=f32)
    tsel = (tio == (len_c - 1)).astype(f32)
    text_last = jnp.zeros((B, TEXT), f32)
    for t in range(T):
        text_last = text_last + tsel[:, t:t + 1] * tf_ref[:, t, :]
    out = jnp.concatenate([u1, u2, u3, pooled, text_last], axis=1)
    mean = jnp.mean(out, axis=0, keepdims=True)
    var = jnp.mean((out - mean) ** 2, axis=0, keepdims=True)
    outn = (out - mean) / jnp.sqrt(var + 1e-5) * gam_ref[...] + bet_ref[...]
    res = jax.nn.sigmoid(jnp.dot(outn, Wcls_ref[...],
                                 preferred_element_type=f32) + bcls_ref[...])
    out_ref[...] = res


def _forward_impl(code_x, divided, neighbors, lens, user, cate, text_features,
                  adj, cate_adj, c_embeddings, n_embeddings, u_embeddings,
                  cate_embeddings, Wg, bg, Wc, bc, Wz, Uz, bz, Wr, Ur, br,
                  Wh, Uh, bh, Wout, bout, Wa, ba, va, emb_gender, emb_age,
                  emb_cluster, bn_gamma, bn_beta, Wcls, bcls, interpret):
    f32 = jnp.float32
    cxa = code_x.reshape(BT, CODE_NUM)
    nba = neighbors.reshape(BT, CODE_NUM)
    Ct = c_embeddings.T
    Nt = n_embeddings.T
    Ut = u_embeddings.T
    CEt = cate_embeddings.T
    cadjT = cate_adj.T
    WgT = Wg.T
    WcT = Wc.T
    dvt = jnp.transpose(divided, (0, 1, 3, 2))

    const2 = lambda s, j: (0, 0)

    xcols = pl.pallas_call(
        _fused_kernel,
        grid=(B + 1, NBLK),
        in_specs=[
            pl.BlockSpec((BT, CODE_NUM), const2),
            pl.BlockSpec((BT, CODE_NUM), const2),
            pl.BlockSpec((1, 1, 3, CODE_NUM),
                         lambda s, j: (jnp.maximum(s - 1, 0), j, 0, 0)),
            pl.BlockSpec((1, 1, CATE_NUM),
                         lambda s, j: (jnp.maximum(s - 1, 0) * T + j, 0, 0)),
            pl.BlockSpec((CODE_NUM, CODE_NUM), const2),
            pl.BlockSpec((CODE_SIZE, CODE_NUM), const2),
            pl.BlockSpec((CODE_SIZE, CODE_NUM), const2),
            pl.BlockSpec((GRAPH_SIZE, CODE_NUM), const2),
            pl.BlockSpec((CODE_SIZE, CATE_NUM), const2),
            pl.BlockSpec((CATE_NUM, CATE_NUM), const2),
            pl.BlockSpec((GRAPH_SIZE, CODE_SIZE), const2),
            pl.BlockSpec((GRAPH_SIZE, 1), const2),
            pl.BlockSpec((GRAPH_SIZE, CODE_SIZE), const2),
            pl.BlockSpec((GRAPH_SIZE, 1), const2),
        ],
        out_specs=pl.BlockSpec(
            (1, 3 * GRAPH_SIZE, 1),
            lambda s, j: (jnp.maximum(s - 1, 0) * T + j, 0, 0)),
        out_shape=jax.ShapeDtypeStruct((BT, 3 * GRAPH_SIZE, 1), f32),
        scratch_shapes=[
            pltpu.VMEM((2 * ZROWS, CODE_NUM), f32),
            pltpu.VMEM((2 * ZROWS, CODE_NUM), f32),
            pltpu.VMEM((GRAPH_SIZE, CODE_NUM), f32),
        ],
        interpret=interpret,
    )(cxa, nba, dvt, cate.reshape(BT, 1, CATE_NUM), adj, Ct, Nt, Ut, CEt,
      cadjT, WgT, bg.reshape(GRAPH_SIZE, 1), WcT, bc.reshape(GRAPH_SIZE, 1))

    xfeat = xcols.reshape(B, T, 3 * GRAPH_SIZE)

    out = pl.pallas_call(
        _head_kernel,
        out_shape=jax.ShapeDtypeStruct((B, OUT_SIZE), f32),
        interpret=interpret,
    )(xfeat, lens.reshape(B, 1).astype(jnp.int32), user.astype(jnp.int32),
      text_features, emb_gender, emb_age, emb_cluster,
      Wz, Uz, bz.reshape(1, HIDDEN), Wr, Ur, br.reshape(1, HIDDEN),
      Wh, Uh, bh.reshape(1, HIDDEN), Wout, bout.reshape(1, HIDDEN),
      Wa, ba.reshape(1, ATT), va.reshape(ATT, 1),
      bn_gamma.reshape(1, -1), bn_beta.reshape(1, -1),
      Wcls, bcls.reshape(1, OUT_SIZE))
    return out


def kernel(code_x, divided, neighbors, lens, user, cate, text_features,
           admission_times, adj, cate_adj, c_embeddings, n_embeddings,
           u_embeddings, cate_embeddings, Wg, bg, Wc, bc, Wz, Uz, bz,
           Wr, Ur, br, Wh, Uh, bh, Wout, bout, Wa, ba, va, emb_gender,
           emb_age, emb_cluster, bn_gamma, bn_beta, Wcls, bcls):
    return _forward_impl(code_x, divided, neighbors, lens, user, cate,
                         text_features, adj, cate_adj, c_embeddings,
                         n_embeddings, u_embeddings, cate_embeddings,
                         Wg, bg, Wc, bc, Wz, Uz, bz, Wr, Ur, br, Wh, Uh, bh,
                         Wout, bout, Wa, ba, va, emb_gender, emb_age,
                         emb_cluster, bn_gamma, bn_beta, Wcls, bcls,
                         interpret=False)


# speedup vs baseline: 1.6295x; 1.6295x over previous
import jax
import jax.numpy as jnp
from jax.experimental import pallas as pl
from jax.experimental.pallas import tpu as pltpu

CODE_NUM = 2048
CODE_SIZE = 48
GRAPH_SIZE = 32
HIDDEN = 64
ATT = 32
CATE_NUM = 128
OUT_SIZE = 2048
TEXT = 300
B = 8
T = 4
BT = B * T
NBLK = 4
BLK = CODE_NUM // NBLK
GCHUNK = 4
NGS = B // GCHUNK


def _mm_kernel(cxa_ref, nba_ref, CNU_ref, adjT_ref, out_ref, zt_ref):
    @pl.when(pl.program_id(0) == 0)
    def _build():
        Ctv = CNU_ref[0:CODE_SIZE, :]
        Ntv = CNU_ref[CODE_SIZE:2 * CODE_SIZE, :]
        for bt in range(BT):
            c = cxa_ref[bt:bt + 1, :]
            n = nba_ref[bt:bt + 1, :]
            zt_ref[bt * CODE_SIZE:(bt + 1) * CODE_SIZE, :] = (
                Ctv * c + Ntv * n).astype(jnp.bfloat16)

    out_ref[...] = jax.lax.dot_general(
        zt_ref[...], adjT_ref[...].astype(jnp.bfloat16), (((1,), (1,)), ((), ())),
        preferred_element_type=jnp.float32).astype(jnp.bfloat16)


def _graph_head_kernel(cx_ref, nb_ref, dv_ref, cate_ref, st_ref, CNU_ref,
                       CE2_ref, WS_ref, lu_ref, tf_ref, emb_ref, gpk_ref,
                       bias_ref, Wa_ref, ba_ref, va_ref, gb_ref,
                       Wcls_ref, bcls_ref, out_ref, xfb_ref):
    f32 = jnp.float32
    gi = pl.program_id(0)
    Ctv = CNU_ref[0:CODE_SIZE, :]
    Ntv = CNU_ref[CODE_SIZE:2 * CODE_SIZE, :]
    Utv = CNU_ref[2 * CODE_SIZE:2 * CODE_SIZE + GRAPH_SIZE, :]
    WgT = WS_ref[:, 0:CODE_SIZE]
    WcT = WS_ref[:, CODE_SIZE:2 * CODE_SIZE]
    bg = WS_ref[:, 2 * CODE_SIZE:2 * CODE_SIZE + 1]
    bc = WS_ref[:, 2 * CODE_SIZE + 1:2 * CODE_SIZE + 2]
    CEt = CE2_ref[0:CODE_SIZE, :]
    cadjT = CE2_ref[CODE_SIZE:CODE_SIZE + CATE_NUM, :]
    for g in range(GCHUNK):
        cx = cx_ref[g]
        nb = nb_ref[g]
        dv = dv_ref[g]
        st = st_ref[g]
        no_prev = None
        for t in range(T):
            c = cx[t:t + 1, :]
            n = nb[t:t + 1, :]
            Sbt = st[t * CODE_SIZE:(t + 1) * CODE_SIZE, :].astype(f32)
            coT = jnp.tanh(jnp.dot(WgT, c * (Ctv + Sbt),
                                   preferred_element_type=f32) + bg)
            noT = jnp.tanh(jnp.dot(WgT, n * (Ntv + Sbt),
                                   preferred_element_type=f32) + bg)
            m1 = dv[t, 0:1, :]
            m23 = dv[t, 1:2, :] + dv[t, 2:3, :]
            pers = jnp.sum(coT * m1, axis=1, keepdims=True)
            if t == 0:
                emer = jnp.zeros((GRAPH_SIZE, 1), f32)
            else:
                candT = m23 * no_prev
                sc = jnp.sum(candT * Utv, axis=0, keepdims=True)
                mx = jnp.max(sc)
                ex = jnp.exp(sc - mx)
                denom = jnp.sum(ex)
                emer = jnp.sum(candT * ex, axis=1, keepdims=True) / denom
            cate_row = cate_ref[g][t:t + 1, :]
            ccenT = CEt * cate_row
            cc2T = jnp.dot(ccenT, cadjT, preferred_element_type=f32)
            caoT = jnp.tanh(jnp.dot(WcT, ccenT + cate_row * cc2T,
                                    preferred_element_type=f32) + bc)
            pool = (jnp.sum(caoT * cate_row, axis=1, keepdims=True)
                    / (jnp.sum(cate_row) + 1e-6))
            bidx = gi * GCHUNK + g
            xfb_ref[pl.ds(bidx, 1), 0:GRAPH_SIZE, t:t + 1] = (
                pers.reshape(1, GRAPH_SIZE, 1))
            xfb_ref[pl.ds(bidx, 1), GRAPH_SIZE:2 * GRAPH_SIZE, t:t + 1] = (
                emer.reshape(1, GRAPH_SIZE, 1))
            xfb_ref[pl.ds(bidx, 1), 2 * GRAPH_SIZE:3 * GRAPH_SIZE, t:t + 1] = (
                pool.reshape(1, GRAPH_SIZE, 1))
            no_prev = noT

    @pl.when(gi == NGS - 1)
    def _head():
        x = xfb_ref[...]
        Wz = gpk_ref[0:96, :]
        Uz = gpk_ref[96:160, :]
        Wr = gpk_ref[160:256, :]
        Ur = gpk_ref[256:320, :]
        Wh = gpk_ref[320:416, :]
        Uh = gpk_ref[416:480, :]
        Wout = gpk_ref[480:544, :]
        bz = bias_ref[0:1, :]
        br = bias_ref[1:2, :]
        bh = bias_ref[2:3, :]
        bout = bias_ref[3:4, :]
        h = jnp.zeros((B, HIDDEN), f32)
        Vs = []
        for t in range(T):
            xt = x[:, :, t]
            z = jax.nn.sigmoid(jnp.dot(xt, Wz, preferred_element_type=f32)
                               + jnp.dot(h, Uz, preferred_element_type=f32)
                               + bz)
            r = jax.nn.sigmoid(jnp.dot(xt, Wr, preferred_element_type=f32)
                               + jnp.dot(h, Ur, preferred_element_type=f32)
                               + br)
            hh = jnp.tanh(jnp.dot(xt, Wh, preferred_element_type=f32)
                          + jnp.dot(r * h, Uh, preferred_element_type=f32)
                          + bh)
            h = (1.0 - z) * h + z * hh
            Vs.append(jnp.tanh(jnp.dot(h, Wout, preferred_element_type=f32)
                               + bout))
        scs = []
        for t in range(T):
            u = jnp.tanh(jnp.dot(Vs[t], Wa_ref[...], preferred_element_type=f32)
                         + ba_ref[...])
            scs.append(jnp.dot(u, va_ref[...], preferred_element_type=f32))
        sc = jnp.concatenate(scs, axis=1)
        len_c = jnp.maximum(lu_ref[:, 3:4], 1)
        tio = jax.lax.broadcasted_iota(jnp.int32, (B, T), 1)
        sc = jnp.where(tio < len_c, sc, -jnp.inf)
        mx = jnp.max(sc, axis=1, keepdims=True)
        ex = jnp.exp(sc - mx)
        al = ex / jnp.sum(ex, axis=1, keepdims=True)
        pooled = jnp.zeros((B, HIDDEN), f32)
        for t in range(T):
            pooled = pooled + al[:, t:t + 1] * Vs[t]
        oh1 = (jax.lax.broadcasted_iota(jnp.int32, (B, 2), 1)
               == lu_ref[:, 0:1]).astype(f32)
        oh2 = (jax.lax.broadcasted_iota(jnp.int32, (B, 9), 1)
               == lu_ref[:, 1:2]).astype(f32)
        oh3 = (jax.lax.broadcasted_iota(jnp.int32, (B, 20), 1)
               == lu_ref[:, 2:3]).astype(f32)
        u1 = jnp.dot(oh1, emb_ref[0:2, :], preferred_element_type=f32)
        u2 = jnp.dot(oh2, emb_ref[2:11, :], preferred_element_type=f32)
        u3 = jnp.dot(oh3, emb_ref[11:31, 0:8], preferred_element_type=f32)
        tsel = (tio == (len_c - 1)).astype(f32)
        text_last = jnp.zeros((B, TEXT), f32)
        for t in range(T):
            text_last = text_last + tsel[:, t:t + 1] * tf_ref[:, t, :]
        out = jnp.concatenate([u1, u2, u3, pooled, text_last], axis=1)
        mean = jnp.mean(out, axis=0, keepdims=True)
        var = jnp.mean((out - mean) ** 2, axis=0, keepdims=True)
        outn = ((out - mean) / jnp.sqrt(var + 1e-5) * gb_ref[0:1, :]
                + gb_ref[1:2, :])
        res = jax.nn.sigmoid(jnp.dot(outn, Wcls_ref[...],
                                     preferred_element_type=f32)
                             + bcls_ref[...])
        out_ref[...] = res


def _forward_impl(code_x, divided, neighbors, lens, user, cate, text_features,
                  adj, cate_adj, c_embeddings, n_embeddings, u_embeddings,
                  cate_embeddings, Wg, bg, Wc, bc, Wz, Uz, bz, Wr, Ur, br,
                  Wh, Uh, bh, Wout, bout, Wa, ba, va, emb_gender, emb_age,
                  emb_cluster, bn_gamma, bn_beta, Wcls, bcls, interpret):
    f32 = jnp.float32
    cxa = code_x.reshape(BT, CODE_NUM)
    nba = neighbors.reshape(BT, CODE_NUM)
    CNU = jnp.concatenate([c_embeddings, n_embeddings, u_embeddings],
                          axis=1).T
    CE2 = jnp.concatenate([cate_embeddings.T, cate_adj.T], axis=0)
    WS = jnp.concatenate([Wg.T, Wc.T, bg.reshape(GRAPH_SIZE, 1),
                          bc.reshape(GRAPH_SIZE, 1)], axis=1)
    lu = jnp.concatenate([user.astype(jnp.int32),
                          lens.reshape(B, 1).astype(jnp.int32)], axis=1)
    emb = jnp.concatenate(
        [emb_gender, emb_age,
         jnp.pad(emb_cluster, ((0, 0), (0, HIDDEN // 4 - HIDDEN // 8)))],
        axis=0)
    gpk = jnp.concatenate([Wz, Uz, Wr, Ur, Wh, Uh, Wout], axis=0)
    bias = jnp.stack([bz, br, bh, bout], axis=0)
    gb = jnp.stack([bn_gamma, bn_beta], axis=0)

    ST = pl.pallas_call(
        _mm_kernel,
        grid=(NBLK,),
        in_specs=[
            pl.BlockSpec((BT, CODE_NUM), lambda j: (0, 0)),
            pl.BlockSpec((BT, CODE_NUM), lambda j: (0, 0)),
            pl.BlockSpec((2 * CODE_SIZE + GRAPH_SIZE, CODE_NUM),
                         lambda j: (0, 0)),
            pl.BlockSpec((BLK, CODE_NUM), lambda j: (j, 0)),
        ],
        out_specs=pl.BlockSpec((BT * CODE_SIZE, BLK), lambda j: (0, j)),
        out_shape=jax.ShapeDtypeStruct((BT * CODE_SIZE, CODE_NUM), jnp.bfloat16),
        scratch_shapes=[pltpu.VMEM((BT * CODE_SIZE, CODE_NUM), jnp.bfloat16)],
        interpret=interpret,
    )(cxa, nba, CNU, adj)

    ST3 = ST.reshape(B, T * CODE_SIZE, CODE_NUM)
    dvt = jnp.transpose(divided, (0, 1, 3, 2))

    cb = lambda b: (b, 0, 0)
    c2 = lambda b: (0, 0)
    out = pl.pallas_call(
        _graph_head_kernel,
        grid=(NGS,),
        in_specs=[
            pl.BlockSpec((GCHUNK, T, CODE_NUM), cb),
            pl.BlockSpec((GCHUNK, T, CODE_NUM), cb),
            pl.BlockSpec((GCHUNK, T, 3, CODE_NUM), lambda b: (b, 0, 0, 0)),
            pl.BlockSpec((GCHUNK, T, CATE_NUM), cb),
            pl.BlockSpec((GCHUNK, T * CODE_SIZE, CODE_NUM), cb),
            pl.BlockSpec((2 * CODE_SIZE + GRAPH_SIZE, CODE_NUM), c2),
            pl.BlockSpec((CODE_SIZE + CATE_NUM, CATE_NUM), c2),
            pl.BlockSpec((GRAPH_SIZE, 2 * CODE_SIZE + 2), c2),
            pl.BlockSpec((B, 4), c2),
            pl.BlockSpec((B, T, TEXT), lambda b: (0, 0, 0)),
            pl.BlockSpec((31, 16), c2),
            pl.BlockSpec((544, HIDDEN), c2),
            pl.BlockSpec((4, HIDDEN), c2),
            pl.BlockSpec((HIDDEN, ATT), c2),
            pl.BlockSpec((1, ATT), c2),
            pl.BlockSpec((ATT, 1), c2),
            pl.BlockSpec((2, 404), c2),
            pl.BlockSpec((404, OUT_SIZE), c2),
            pl.BlockSpec((1, OUT_SIZE), c2),
        ],
        out_specs=pl.BlockSpec((B, OUT_SIZE), c2),
        out_shape=jax.ShapeDtypeStruct((B, OUT_SIZE), f32),
        scratch_shapes=[pltpu.VMEM((B, 3 * GRAPH_SIZE, T), f32)],
        interpret=interpret,
    )(code_x, neighbors, dvt, cate, ST3, CNU, CE2, WS, lu, text_features,
      emb, gpk, bias, Wa, ba.reshape(1, ATT), va.reshape(ATT, 1), gb,
      Wcls, bcls.reshape(1, OUT_SIZE))
    return out


def kernel(code_x, divided, neighbors, lens, user, cate, text_features,
           admission_times, adj, cate_adj, c_embeddings, n_embeddings,
           u_embeddings, cate_embeddings, Wg, bg, Wc, bc, Wz, Uz, bz,
           Wr, Ur, br, Wh, Uh, bh, Wout, bout, Wa, ba, va, emb_gender,
           emb_age, emb_cluster, bn_gamma, bn_beta, Wcls, bcls):
    return _forward_impl(code_x, divided, neighbors, lens, user, cate,
                         text_features, adj, cate_adj, c_embeddings,
                         n_embeddings, u_embeddings, cate_embeddings,
                         Wg, bg, Wc, bc, Wz, Uz, bz, Wr, Ur, br, Wh, Uh, bh,
                         Wout, bout, Wa, ba, va, emb_gender, emb_age,
                         emb_cluster, bn_gamma, bn_beta, Wcls, bcls,
                         interpret=False)
